# baseline (speedup 1.0000x reference)
import functools

import jax
import jax.numpy as jnp
from jax import lax
from jax.experimental import pallas as pl
from jax.experimental.pallas import tpu as pltpu
from jax.experimental.pallas import tpu_sc as plsc

_LANES = 16
_CHUNK = 8


def _make_sc_kernel(B, L, D, pe_rows):
    info = plsc.get_sparse_core_info()
    nw = info.num_cores * info.num_subcores
    rows_per_w = L // nw
    n_chunks = rows_per_w // _CHUNK
    mesh = plsc.VectorSubcoreMesh(core_axis_name="c", subcore_axis_name="s")
    cd = _CHUNK * D

    @functools.partial(
        pl.kernel,
        mesh=mesh,
        out_type=jax.ShapeDtypeStruct((B, L * D), jnp.float32),
        scratch_types=[
            pltpu.VMEM((cd,), jnp.float32),
            pltpu.VMEM((B, cd), jnp.float32),
            pltpu.SemaphoreType.DMA,
        ],
    )
    def k(x_hbm, pe_hbm, out_hbm, pe_v, xb_v, sem):
        wid = lax.axis_index("s") * info.num_cores + lax.axis_index("c")
        base = wid * rows_per_w * D

        def chunk_body(ci, carry):
            off = base + ci * cd
            cp_pe = pltpu.make_async_copy(pe_hbm.at[pl.ds(off, cd)], pe_v, sem)
            cp_pe.start()
            cps = []
            for b in range(B):
                cp = pltpu.make_async_copy(
                    x_hbm.at[b, pl.ds(off, cd)], xb_v.at[b], sem)
                cp.start()
                cps.append(cp)
            cp_pe.wait()
            for cp in cps:
                cp.wait()

            def row_body(r, c2):
                rb = r * D
                for o in range(D // _LANES):
                    sl = pl.ds(rb + o * _LANES, _LANES)
                    v = pe_v[sl]
                    for b in range(B):
                        plsc.addupdate(xb_v.at[b, sl], v)
                return c2

            lax.fori_loop(0, _CHUNK, row_body, 0)
            for b in range(B):
                pltpu.sync_copy(xb_v.at[b], out_hbm.at[b, pl.ds(off, cd)])
            return carry

        lax.fori_loop(0, n_chunks, chunk_body, 0)

    return k


def kernel(x, pe_table):
    B, L, D = x.shape
    k = _make_sc_kernel(B, L, D, pe_table.shape[0])
    out = k(x.reshape(B, L * D), pe_table.reshape(-1))
    return out.reshape(B, L, D)


# speedup vs baseline: 1.1219x; 1.1219x over previous
import functools

import jax
import jax.numpy as jnp
from jax import lax
from jax.experimental import pallas as pl
from jax.experimental.pallas import tpu as pltpu
from jax.experimental.pallas import tpu_sc as plsc

_LANES = 16
_CHUNK = 4
_NBUF = 4


def _make_sc_kernel(B, L, D):
    info = plsc.get_sparse_core_info()
    nw = info.num_cores * info.num_subcores
    rows_per_w = L // nw
    n_chunks = rows_per_w // _CHUNK
    mesh = plsc.VectorSubcoreMesh(core_axis_name="c", subcore_axis_name="s")
    cd = _CHUNK * D

    @functools.partial(
        pl.kernel,
        mesh=mesh,
        out_type=jax.ShapeDtypeStruct((B, L * D), jnp.float32),
        scratch_types=[
            pltpu.VMEM((_NBUF, cd), jnp.float32),
            pltpu.VMEM((_NBUF, B, cd), jnp.float32),
            [pltpu.SemaphoreType.DMA] * _NBUF,
            [pltpu.SemaphoreType.DMA] * _NBUF,
        ],
    )
    def k(x_hbm, pe_hbm, out_hbm, pe_v, xb_v, sin, sout):
        wid = lax.axis_index("s") * info.num_cores + lax.axis_index("c")
        base = wid * rows_per_w * D

        def in_copies(ci, slot):
            off = base + ci * cd
            cps = [pltpu.make_async_copy(
                pe_hbm.at[pl.ds(off, cd)], pe_v.at[slot], sin[slot])]
            for b in range(B):
                cps.append(pltpu.make_async_copy(
                    x_hbm.at[b, pl.ds(off, cd)], xb_v.at[slot, b], sin[slot]))
            return cps

        def out_copies(ci, slot):
            off = base + ci * cd
            return [pltpu.make_async_copy(
                xb_v.at[slot, b], out_hbm.at[b, pl.ds(off, cd)], sout[slot])
                for b in range(B)]

        def start(cps):
            for cp in cps:
                cp.start()

        def wait(cps):
            for cp in cps:
                cp.wait()

        start(in_copies(0, 0))
        start(in_copies(1, 1))

        def group_body(j, carry):
            for s in range(_NBUF):
                ci = j * _NBUF + s
                wait(in_copies(ci, s))

                def row_body(r, c2):
                    rb = r * D
                    for o in range(D // _LANES):
                        sl = pl.ds(rb + o * _LANES, _LANES)
                        v = pe_v[s, sl]
                        for b in range(B):
                            plsc.addupdate(xb_v.at[s, b, sl], v)
                    return c2

                lax.fori_loop(0, _CHUNK, row_body, 0)
                start(out_copies(ci, s))

                t = (s + 2) % _NBUF

                @pl.when(ci >= 2)
                def _():
                    wait(out_copies(ci - 2, t))

                @pl.when(ci + 2 < n_chunks)
                def _():
                    start(in_copies(ci + 2, t))

            return carry

        lax.fori_loop(0, n_chunks // _NBUF, group_body, 0)
        wait(out_copies(n_chunks - 2, (n_chunks - 2) % _NBUF))
        wait(out_copies(n_chunks - 1, (n_chunks - 1) % _NBUF))

    return k


def kernel(x, pe_table):
    B, L, D = x.shape
    k = _make_sc_kernel(B, L, D)
    out = k(x.reshape(B, L * D), pe_table.reshape(-1))
    return out.reshape(B, L, D)


# speedup vs baseline: 1.1698x; 1.0427x over previous
import functools

import jax
import jax.numpy as jnp
from jax import lax
from jax.experimental import pallas as pl
from jax.experimental.pallas import tpu as pltpu
from jax.experimental.pallas import tpu_sc as plsc

_LANES = 16
_CHUNK = 4
_NBUF = 4


def _make_sc_kernel(B, L, D):
    info = plsc.get_sparse_core_info()
    nw = info.num_cores * info.num_subcores
    rows_per_w = L // nw
    n_chunks = rows_per_w // _CHUNK
    mesh = plsc.VectorSubcoreMesh(core_axis_name="c", subcore_axis_name="s")
    cd = _CHUNK * D

    @functools.partial(
        pl.kernel,
        mesh=mesh,
        out_type=jax.ShapeDtypeStruct((B, L * D), jnp.float32),
        scratch_types=[
            pltpu.VMEM((_NBUF, cd), jnp.float32),
            pltpu.VMEM((_NBUF, B, cd), jnp.float32),
            [pltpu.SemaphoreType.DMA] * _NBUF,
            [pltpu.SemaphoreType.DMA] * _NBUF,
        ],
    )
    def k(x_hbm, pe_hbm, out_hbm, pe_v, xb_v, sin, sout):
        wid = lax.axis_index("s") * info.num_cores + lax.axis_index("c")
        base = wid * rows_per_w * D

        def in_copies(ci, slot):
            off = base + ci * cd
            cps = [pltpu.make_async_copy(
                pe_hbm.at[pl.ds(off, cd)], pe_v.at[slot], sin[slot])]
            for b in range(B):
                cps.append(pltpu.make_async_copy(
                    x_hbm.at[b, pl.ds(off, cd)], xb_v.at[slot, b], sin[slot]))
            return cps

        def out_copies(ci, slot):
            off = base + ci * cd
            return [pltpu.make_async_copy(
                xb_v.at[slot, b], out_hbm.at[b, pl.ds(off, cd)], sout[slot])
                for b in range(B)]

        def start(cps):
            for cp in cps:
                cp.start()

        def wait(cps):
            for cp in cps:
                cp.wait()

        start(in_copies(0, 0))
        start(in_copies(1, 1))

        def group_body(j, carry):
            for s in range(_NBUF):
                ci = j * _NBUF + s
                wait(in_copies(ci, s))

                @plsc.parallel_loop(0, cd // _LANES, unroll=8)
                def _(i):
                    sl = pl.ds(i * _LANES, _LANES)
                    v = pe_v[s, sl]
                    for b in range(B):
                        plsc.addupdate(xb_v.at[s, b, sl], v)
                start(out_copies(ci, s))

                t = (s + 2) % _NBUF

                @pl.when(ci >= 2)
                def _():
                    wait(out_copies(ci - 2, t))

                @pl.when(ci + 2 < n_chunks)
                def _():
                    start(in_copies(ci + 2, t))

            return carry

        lax.fori_loop(0, n_chunks // _NBUF, group_body, 0)
        wait(out_copies(n_chunks - 2, (n_chunks - 2) % _NBUF))
        wait(out_copies(n_chunks - 1, (n_chunks - 1) % _NBUF))

    return k


def kernel(x, pe_table):
    B, L, D = x.shape
    k = _make_sc_kernel(B, L, D)
    out = k(x.reshape(B, L * D), pe_table.reshape(-1))
    return out.reshape(B, L, D)


# speedup vs baseline: 3.2081x; 2.7424x over previous
import functools

import jax
import jax.numpy as jnp
from jax import lax
from jax.experimental import pallas as pl
from jax.experimental.pallas import tpu as pltpu
from jax.experimental.pallas import tpu_sc as plsc

_LANES = 16
_CHUNK = 8
_NBUF = 3


def _make_sc_kernel(B, L, D):
    info = plsc.get_sparse_core_info()
    nw = info.num_cores * info.num_subcores
    rows_per_w = L // nw
    n_chunks = rows_per_w // _CHUNK
    mesh = plsc.VectorSubcoreMesh(core_axis_name="c", subcore_axis_name="s")
    vecs_per_row = D // _LANES

    @functools.partial(
        pl.kernel,
        mesh=mesh,
        out_type=jax.ShapeDtypeStruct((B, L, D), jnp.float32),
        scratch_types=[
            [pltpu.VMEM((_CHUNK, D), jnp.float32) for _ in range(_NBUF)],
            [[pltpu.VMEM((_CHUNK, D), jnp.float32) for _ in range(B)]
             for _ in range(_NBUF)],
            [pltpu.SemaphoreType.DMA] * _NBUF,
            [pltpu.SemaphoreType.DMA] * _NBUF,
        ],
        compiler_params=pltpu.CompilerParams(use_tc_tiling_on_sc=True),
    )
    def k(x_hbm, pe_hbm, out_hbm, pe_v, xb_v, sin, sout):
        wid = lax.axis_index("s") * info.num_cores + lax.axis_index("c")
        base = wid * rows_per_w

        def in_copies(ci):
            s = ci % _NBUF
            r0 = base + ci * _CHUNK
            cps = [pltpu.make_async_copy(
                pe_hbm.at[pl.ds(r0, _CHUNK), :], pe_v[s], sin[s])]
            for b in range(B):
                cps.append(pltpu.make_async_copy(
                    x_hbm.at[b, pl.ds(r0, _CHUNK), :], xb_v[s][b], sin[s]))
            return cps

        def out_copies(ci):
            s = ci % _NBUF
            r0 = base + ci * _CHUNK
            return [pltpu.make_async_copy(
                xb_v[s][b], out_hbm.at[b, pl.ds(r0, _CHUNK), :], sout[s])
                for b in range(B)]

        def start(cps):
            for cp in cps:
                cp.start()

        def wait(cps):
            for cp in cps:
                cp.wait()

        start(in_copies(0))
        for ci in range(n_chunks):
            if ci + 1 < n_chunks:
                if ci - 2 >= 0:
                    wait(out_copies(ci - 2))
                start(in_copies(ci + 1))
            wait(in_copies(ci))
            s = ci % _NBUF

            @plsc.parallel_loop(0, _CHUNK * vecs_per_row, unroll=8)
            def _(i):
                r = i // vecs_per_row
                sl = pl.ds((i % vecs_per_row) * _LANES, _LANES)
                v = pe_v[s][r, sl]
                for b in range(B):
                    plsc.addupdate(xb_v[s][b].at[r, sl], v)

            start(out_copies(ci))

        for ci in range(n_chunks - 3, n_chunks):
            wait(out_copies(ci))

    return k


def kernel(x, pe_table):
    B, L, D = x.shape
    k = _make_sc_kernel(B, L, D)
    return k(x, pe_table)
